# baseline (speedup 1.0000x reference)
import functools

import jax
import jax.numpy as jnp
from jax import lax
from jax.experimental import pallas as pl
from jax.experimental.pallas import tpu as pltpu
from jax.experimental.pallas import tpu_sc as plsc

N = 10000
E = 160000
D = 128

NC = 2
NS = 16
NW = NC * NS

CHUNK = 128
NCH0 = 16
NCH1 = 64
NSTAGE = 32
TOT_CHUNKS = NS * (NCH0 + NCH1)
E_PAD = TOT_CHUNKS * CHUNK

N_ACC = 10240
ZROWS = N_ACC // NS
ZSTEPS = ZROWS // CHUNK

NB = 2
SLACK = 1


def _sc_aggregate_body(nf_hbm, src_hbm, dst_hbm, zeros_hbm, out_hbm,
                       sidx, didx, rows, acc_sh, *sems):
    gsem = sems[:NB]
    ssem = sems[NB:]
    cid = lax.axis_index("c")
    sid = lax.axis_index("s")

    def edge_loop(base, nchunks):
        pltpu.sync_copy(src_hbm.at[pl.ds(base, nchunks)],
                        sidx.at[pl.ds(0, nchunks)])
        pltpu.sync_copy(dst_hbm.at[pl.ds(base, nchunks)],
                        didx.at[pl.ds(0, nchunks)])

        def fire_gather(k, p):
            return pltpu.async_copy(nf_hbm.at[sidx.at[k]], rows.at[p],
                                    gsem[p])

        def wait_gather(k, p):
            pltpu.make_async_copy(nf_hbm.at[sidx.at[k]], rows.at[p],
                                  gsem[p]).wait()

        def fire_scatter(k, p):
            return pltpu.async_copy(rows.at[p], acc_sh.at[didx.at[k]],
                                    ssem[p], add=True)

        def wait_scatter(p):
            pltpu.make_async_copy(rows.at[p], acc_sh.at[didx.at[0]],
                                  ssem[p]).wait()

        for p in range(NB):
            fire_gather(p, p)

        def group(g, _):
            for p in range(NB):
                k = g * NB + p
                wait_gather(k, p)
                fire_scatter(k, p)
                q = (p + NB - SLACK) % NB
                j = k + NB - SLACK

                @pl.when(k >= SLACK)
                def _():
                    wait_scatter(q)

                @pl.when(jnp.logical_and(k >= SLACK, j < nchunks))
                def _():
                    fire_gather(j, q)
            return ()

        lax.fori_loop(0, nchunks // NB, group, ())
        for t in range(SLACK):
            wait_scatter((nchunks - SLACK + t) % NB)

    pltpu.sync_copy(zeros_hbm, rows.at[0])
    for i in range(ZSTEPS):
        pltpu.sync_copy(rows.at[0],
                        acc_sh.at[pl.ds(sid * ZROWS + i * CHUNK, CHUNK)])
    plsc.subcore_barrier()

    def run_core(base, total):
        done = 0
        while done < total:
            step = min(NSTAGE, total - done)
            edge_loop(base + done, step)
            done += step

    if NCH0 > 0:
        @pl.when(cid == 0)
        def _():
            run_core(sid * NCH0, NCH0)

    if NCH1 > 0:
        @pl.when(cid == 1)
        def _():
            run_core(NS * NCH0 + sid * NCH1, NCH1)

    plsc.subcore_barrier()
    ods = [None] * NB
    for i in range(ZSTEPS):
        p = i % NB
        r0 = sid * ZROWS + i * CHUNK
        if ods[p] is not None:
            ods[p].wait()
        pltpu.sync_copy(acc_sh.at[pl.ds(r0, CHUNK)], rows.at[p])
        ods[p] = pltpu.async_copy(rows.at[p],
                                  out_hbm.at[cid, pl.ds(r0, CHUNK)], gsem[p])
    for p in range(NB):
        if ods[p] is not None:
            ods[p].wait()


@functools.cache
def _sc_aggregate():
    return pl.kernel(
        _sc_aggregate_body,
        out_type=jax.ShapeDtypeStruct((NC, N_ACC, D), jnp.float32),
        mesh=plsc.VectorSubcoreMesh(core_axis_name="c", subcore_axis_name="s",
                                    num_cores=NC, num_subcores=NS),
        scratch_types=[
            pltpu.VMEM((NSTAGE, CHUNK), jnp.int32),
            pltpu.VMEM((NSTAGE, CHUNK), jnp.int32),
            pltpu.VMEM((NB, CHUNK, D), jnp.float32),
            pltpu.VMEM_SHARED((N_ACC, D), jnp.float32),
        ] + [pltpu.SemaphoreType.DMA] * (2 * NB),
    )


R_MLP = 2000


def _mlp_body(x_ref, p0_ref, p1_ref, eps_ref, w1_ref, b1_ref, w2_ref, b2_ref,
              w3_ref, b3_ref, h_ref):
    agg = p0_ref[0] + p1_ref[0]
    hb = (1.0 + eps_ref[0, 0]) * x_ref[...] + agg
    h1 = jnp.maximum(jnp.dot(hb, w1_ref[...],
                             preferred_element_type=jnp.float32) + b1_ref[...], 0.0)
    h2 = jnp.maximum(jnp.dot(h1, w2_ref[...],
                             preferred_element_type=jnp.float32) + b2_ref[...], 0.0)
    h3 = jnp.maximum(jnp.dot(h2, w3_ref[...],
                             preferred_element_type=jnp.float32) + b3_ref[...], 0.0)
    h_ref[...] = h3


_mlp = pl.pallas_call(
    _mlp_body,
    grid=(N // R_MLP,),
    in_specs=[
        pl.BlockSpec((R_MLP, D), lambda i: (i, 0)),
        pl.BlockSpec((1, R_MLP, D), lambda i: (0, i, 0)),
        pl.BlockSpec((1, R_MLP, D), lambda i: (1, i, 0)),
        pl.BlockSpec((1, 1), lambda i: (0, 0)),
        pl.BlockSpec((D, 16), lambda i: (0, 0)),
        pl.BlockSpec((1, 16), lambda i: (0, 0)),
        pl.BlockSpec((16, 16), lambda i: (0, 0)),
        pl.BlockSpec((1, 16), lambda i: (0, 0)),
        pl.BlockSpec((16, 1), lambda i: (0, 0)),
        pl.BlockSpec((1, 1), lambda i: (0, 0)),
    ],
    out_specs=pl.BlockSpec((R_MLP, 1), lambda i: (i, 0)),
    out_shape=jax.ShapeDtypeStruct((N, 1), jnp.float32),
)


BI = 400
BJ = 10000


def _outer_body(hi_ref, hj_ref, out_ref):
    out_ref[...] = hi_ref[...] * hj_ref[...]


_outer = pl.pallas_call(
    _outer_body,
    grid=(N // BI, N // BJ),
    in_specs=[
        pl.BlockSpec((BI, 1), lambda i, j: (i, 0)),
        pl.BlockSpec((1, BJ), lambda i, j: (0, j)),
    ],
    out_specs=pl.BlockSpec((BI, BJ), lambda i, j: (i, j)),
    out_shape=jax.ShapeDtypeStruct((N, N), jnp.float32),
)


@jax.jit
def kernel(node_feats, edge_idx, eps, W1, b1, W2, b2, W3, b3):
    src = jnp.concatenate(
        [edge_idx[0], jnp.zeros((E_PAD - E,), jnp.int32)]).reshape(TOT_CHUNKS, CHUNK)
    dst = jnp.concatenate(
        [edge_idx[1], jnp.full((E_PAD - E,), N, jnp.int32)]).reshape(TOT_CHUNKS, CHUNK)
    zeros = jnp.zeros((CHUNK, D), jnp.float32)

    partials = _sc_aggregate()(node_feats, src, dst, zeros)

    h = _mlp(node_feats, partials, partials,
             (1.0 * eps).reshape(1, 1),
             W1.T, b1.reshape(1, 16),
             W2.T, b2.reshape(1, 16),
             W3.T, b3.reshape(1, 1))

    return _outer(h, h.reshape(1, N))


# speedup vs baseline: 1.1539x; 1.1539x over previous
import functools

import jax
import jax.numpy as jnp
from jax import lax
from jax.experimental import pallas as pl
from jax.experimental.pallas import tpu as pltpu
from jax.experimental.pallas import tpu_sc as plsc

N = 10000
E = 160000
D = 128

NC = 2
NS = 16
NW = NC * NS

CHUNK = 128
NCH0 = 64
NCH1 = 16
NSTAGE = 32
TOT_CHUNKS = NS * (NCH0 + NCH1)
E_PAD = TOT_CHUNKS * CHUNK

N_ACC = 10240
ZROWS = N_ACC // NS
ZSTEPS = ZROWS // CHUNK

NB = 2
SLACK = 1


def _sc_aggregate_body(nf_hbm, src_hbm, dst_hbm, zeros_hbm, out_hbm,
                       sidx, didx, rows, acc_sh, *sems):
    gsem = sems[:NB]
    ssem = sems[NB:]
    cid = lax.axis_index("c")
    sid = lax.axis_index("s")

    def edge_loop(base, nchunks):
        pltpu.sync_copy(src_hbm.at[pl.ds(base, nchunks)],
                        sidx.at[pl.ds(0, nchunks)])
        pltpu.sync_copy(dst_hbm.at[pl.ds(base, nchunks)],
                        didx.at[pl.ds(0, nchunks)])

        def fire_gather(k, p):
            return pltpu.async_copy(nf_hbm.at[sidx.at[k]], rows.at[p],
                                    gsem[p])

        def wait_gather(k, p):
            pltpu.make_async_copy(nf_hbm.at[sidx.at[k]], rows.at[p],
                                  gsem[p]).wait()

        def fire_scatter(k, p):
            return pltpu.async_copy(rows.at[p], acc_sh.at[didx.at[k]],
                                    ssem[p], add=True)

        def wait_scatter(p):
            pltpu.make_async_copy(rows.at[p], acc_sh.at[didx.at[0]],
                                  ssem[p]).wait()

        for p in range(NB):
            fire_gather(p, p)

        def group(g, _):
            for p in range(NB):
                k = g * NB + p
                wait_gather(k, p)
                fire_scatter(k, p)
                q = (p + NB - SLACK) % NB
                j = k + NB - SLACK

                @pl.when(k >= SLACK)
                def _():
                    wait_scatter(q)

                @pl.when(jnp.logical_and(k >= SLACK, j < nchunks))
                def _():
                    fire_gather(j, q)
            return ()

        lax.fori_loop(0, nchunks // NB, group, ())
        for t in range(SLACK):
            wait_scatter((nchunks - SLACK + t) % NB)

    pltpu.sync_copy(zeros_hbm, rows.at[0])
    for i in range(ZSTEPS):
        pltpu.sync_copy(rows.at[0],
                        acc_sh.at[pl.ds(sid * ZROWS + i * CHUNK, CHUNK)])
    plsc.subcore_barrier()

    def run_core(base, total):
        done = 0
        while done < total:
            step = min(NSTAGE, total - done)
            edge_loop(base + done, step)
            done += step

    if NCH0 > 0:
        @pl.when(cid == 0)
        def _():
            run_core(sid * NCH0, NCH0)

    if NCH1 > 0:
        @pl.when(cid == 1)
        def _():
            run_core(NS * NCH0 + sid * NCH1, NCH1)

    plsc.subcore_barrier()
    ods = [None] * NB
    for i in range(ZSTEPS):
        p = i % NB
        r0 = sid * ZROWS + i * CHUNK
        if ods[p] is not None:
            ods[p].wait()
        pltpu.sync_copy(acc_sh.at[pl.ds(r0, CHUNK)], rows.at[p])
        ods[p] = pltpu.async_copy(rows.at[p],
                                  out_hbm.at[cid, pl.ds(r0, CHUNK)], gsem[p])
    for p in range(NB):
        if ods[p] is not None:
            ods[p].wait()


@functools.cache
def _sc_aggregate():
    return pl.kernel(
        _sc_aggregate_body,
        out_type=jax.ShapeDtypeStruct((NC, N_ACC, D), jnp.float32),
        mesh=plsc.VectorSubcoreMesh(core_axis_name="c", subcore_axis_name="s",
                                    num_cores=NC, num_subcores=NS),
        scratch_types=[
            pltpu.VMEM((NSTAGE, CHUNK), jnp.int32),
            pltpu.VMEM((NSTAGE, CHUNK), jnp.int32),
            pltpu.VMEM((NB, CHUNK, D), jnp.float32),
            pltpu.VMEM_SHARED((N_ACC, D), jnp.float32),
        ] + [pltpu.SemaphoreType.DMA] * (2 * NB),
    )


R_MLP = 2000


def _mlp_body(x_ref, p0_ref, p1_ref, eps_ref, w1_ref, b1_ref, w2_ref, b2_ref,
              w3_ref, b3_ref, h_ref):
    agg = p0_ref[0] + p1_ref[0]
    hb = (1.0 + eps_ref[0, 0]) * x_ref[...] + agg
    h1 = jnp.maximum(jnp.dot(hb, w1_ref[...],
                             preferred_element_type=jnp.float32) + b1_ref[...], 0.0)
    h2 = jnp.maximum(jnp.dot(h1, w2_ref[...],
                             preferred_element_type=jnp.float32) + b2_ref[...], 0.0)
    h3 = jnp.maximum(jnp.dot(h2, w3_ref[...],
                             preferred_element_type=jnp.float32) + b3_ref[...], 0.0)
    h_ref[...] = h3


_mlp = pl.pallas_call(
    _mlp_body,
    grid=(N // R_MLP,),
    in_specs=[
        pl.BlockSpec((R_MLP, D), lambda i: (i, 0)),
        pl.BlockSpec((1, R_MLP, D), lambda i: (0, i, 0)),
        pl.BlockSpec((1, R_MLP, D), lambda i: (1, i, 0)),
        pl.BlockSpec((1, 1), lambda i: (0, 0)),
        pl.BlockSpec((D, 16), lambda i: (0, 0)),
        pl.BlockSpec((1, 16), lambda i: (0, 0)),
        pl.BlockSpec((16, 16), lambda i: (0, 0)),
        pl.BlockSpec((1, 16), lambda i: (0, 0)),
        pl.BlockSpec((16, 1), lambda i: (0, 0)),
        pl.BlockSpec((1, 1), lambda i: (0, 0)),
    ],
    out_specs=pl.BlockSpec((R_MLP, 1), lambda i: (i, 0)),
    out_shape=jax.ShapeDtypeStruct((N, 1), jnp.float32),
)


BI = 400
BJ = 10000


def _outer_body(hi_ref, hj_ref, out_ref):
    out_ref[...] = hi_ref[...] * hj_ref[...]


_outer = pl.pallas_call(
    _outer_body,
    grid=(N // BI, N // BJ),
    in_specs=[
        pl.BlockSpec((BI, 1), lambda i, j: (i, 0)),
        pl.BlockSpec((1, BJ), lambda i, j: (0, j)),
    ],
    out_specs=pl.BlockSpec((BI, BJ), lambda i, j: (i, j)),
    out_shape=jax.ShapeDtypeStruct((N, N), jnp.float32),
)


@jax.jit
def kernel(node_feats, edge_idx, eps, W1, b1, W2, b2, W3, b3):
    src = jnp.concatenate(
        [edge_idx[0], jnp.zeros((E_PAD - E,), jnp.int32)]).reshape(TOT_CHUNKS, CHUNK)
    dst = jnp.concatenate(
        [edge_idx[1], jnp.full((E_PAD - E,), N, jnp.int32)]).reshape(TOT_CHUNKS, CHUNK)
    zeros = jnp.zeros((CHUNK, D), jnp.float32)

    partials = _sc_aggregate()(node_feats, src, dst, zeros)

    h = _mlp(node_feats, partials, partials,
             (1.0 * eps).reshape(1, 1),
             W1.T, b1.reshape(1, 16),
             W2.T, b2.reshape(1, 16),
             W3.T, b3.reshape(1, 1))

    return _outer(h, h.reshape(1, N))


# speedup vs baseline: 1.2309x; 1.0666x over previous
import functools

import jax
import jax.numpy as jnp
from jax import lax
from jax.experimental import pallas as pl
from jax.experimental.pallas import tpu as pltpu
from jax.experimental.pallas import tpu_sc as plsc

N = 10000
E = 160000
D = 128

NC = 2
NS = 16
NW = NC * NS

CHUNK = 128
NCH0 = 72
NCH1 = 8
NSTAGE = 32
TOT_CHUNKS = NS * (NCH0 + NCH1)
E_PAD = TOT_CHUNKS * CHUNK

N_ACC = 10240
ZROWS = N_ACC // NS
ZSTEPS = ZROWS // CHUNK

NB = 2
SLACK = 1


def _sc_aggregate_body(nf_hbm, src_hbm, dst_hbm, zeros_hbm, out_hbm,
                       sidx, didx, rows, acc_sh, *sems):
    gsem = sems[:NB]
    ssem = sems[NB:]
    cid = lax.axis_index("c")
    sid = lax.axis_index("s")

    def edge_loop(base, nchunks):
        pltpu.sync_copy(src_hbm.at[pl.ds(base, nchunks)],
                        sidx.at[pl.ds(0, nchunks)])
        pltpu.sync_copy(dst_hbm.at[pl.ds(base, nchunks)],
                        didx.at[pl.ds(0, nchunks)])

        def fire_gather(k, p):
            return pltpu.async_copy(nf_hbm.at[sidx.at[k]], rows.at[p],
                                    gsem[p])

        def wait_gather(k, p):
            pltpu.make_async_copy(nf_hbm.at[sidx.at[k]], rows.at[p],
                                  gsem[p]).wait()

        def fire_scatter(k, p):
            return pltpu.async_copy(rows.at[p], acc_sh.at[didx.at[k]],
                                    ssem[p], add=True)

        def wait_scatter(p):
            pltpu.make_async_copy(rows.at[p], acc_sh.at[didx.at[0]],
                                  ssem[p]).wait()

        for p in range(NB):
            fire_gather(p, p)

        def group(g, _):
            for p in range(NB):
                k = g * NB + p
                wait_gather(k, p)
                fire_scatter(k, p)
                q = (p + NB - SLACK) % NB
                j = k + NB - SLACK

                @pl.when(k >= SLACK)
                def _():
                    wait_scatter(q)

                @pl.when(jnp.logical_and(k >= SLACK, j < nchunks))
                def _():
                    fire_gather(j, q)
            return ()

        lax.fori_loop(0, nchunks // NB, group, ())
        for t in range(SLACK):
            wait_scatter((nchunks - SLACK + t) % NB)

    pltpu.sync_copy(zeros_hbm, rows.at[0])
    for i in range(ZSTEPS):
        pltpu.sync_copy(rows.at[0],
                        acc_sh.at[pl.ds(sid * ZROWS + i * CHUNK, CHUNK)])
    plsc.subcore_barrier()

    def run_core(base, total):
        done = 0
        while done < total:
            step = min(NSTAGE, total - done)
            edge_loop(base + done, step)
            done += step

    if NCH0 > 0:
        @pl.when(cid == 0)
        def _():
            run_core(sid * NCH0, NCH0)

    if NCH1 > 0:
        @pl.when(cid == 1)
        def _():
            run_core(NS * NCH0 + sid * NCH1, NCH1)

    plsc.subcore_barrier()
    ods = [None] * NB
    for i in range(ZSTEPS):
        p = i % NB
        r0 = sid * ZROWS + i * CHUNK
        if ods[p] is not None:
            ods[p].wait()
        pltpu.sync_copy(acc_sh.at[pl.ds(r0, CHUNK)], rows.at[p])
        ods[p] = pltpu.async_copy(rows.at[p],
                                  out_hbm.at[cid, pl.ds(r0, CHUNK)], gsem[p])
    for p in range(NB):
        if ods[p] is not None:
            ods[p].wait()


@functools.cache
def _sc_aggregate():
    return pl.kernel(
        _sc_aggregate_body,
        out_type=jax.ShapeDtypeStruct((NC, N_ACC, D), jnp.float32),
        mesh=plsc.VectorSubcoreMesh(core_axis_name="c", subcore_axis_name="s",
                                    num_cores=NC, num_subcores=NS),
        scratch_types=[
            pltpu.VMEM((NSTAGE, CHUNK), jnp.int32),
            pltpu.VMEM((NSTAGE, CHUNK), jnp.int32),
            pltpu.VMEM((NB, CHUNK, D), jnp.float32),
            pltpu.VMEM_SHARED((N_ACC, D), jnp.float32),
        ] + [pltpu.SemaphoreType.DMA] * (2 * NB),
    )


R_MLP = 2000


def _mlp_body(x_ref, p0_ref, p1_ref, eps_ref, w1_ref, b1_ref, w2_ref, b2_ref,
              w3_ref, b3_ref, h_ref):
    agg = p0_ref[0] + p1_ref[0]
    hb = (1.0 + eps_ref[0, 0]) * x_ref[...] + agg
    h1 = jnp.maximum(jnp.dot(hb, w1_ref[...],
                             preferred_element_type=jnp.float32) + b1_ref[...], 0.0)
    h2 = jnp.maximum(jnp.dot(h1, w2_ref[...],
                             preferred_element_type=jnp.float32) + b2_ref[...], 0.0)
    h3 = jnp.maximum(jnp.dot(h2, w3_ref[...],
                             preferred_element_type=jnp.float32) + b3_ref[...], 0.0)
    h_ref[...] = h3


_mlp = pl.pallas_call(
    _mlp_body,
    grid=(N // R_MLP,),
    in_specs=[
        pl.BlockSpec((R_MLP, D), lambda i: (i, 0)),
        pl.BlockSpec((1, R_MLP, D), lambda i: (0, i, 0)),
        pl.BlockSpec((1, R_MLP, D), lambda i: (1, i, 0)),
        pl.BlockSpec((1, 1), lambda i: (0, 0)),
        pl.BlockSpec((D, 16), lambda i: (0, 0)),
        pl.BlockSpec((1, 16), lambda i: (0, 0)),
        pl.BlockSpec((16, 16), lambda i: (0, 0)),
        pl.BlockSpec((1, 16), lambda i: (0, 0)),
        pl.BlockSpec((16, 1), lambda i: (0, 0)),
        pl.BlockSpec((1, 1), lambda i: (0, 0)),
    ],
    out_specs=pl.BlockSpec((R_MLP, 1), lambda i: (i, 0)),
    out_shape=jax.ShapeDtypeStruct((N, 1), jnp.float32),
)


BI = 400
BJ = 10000


def _outer_body(hi_ref, hj_ref, out_ref):
    out_ref[...] = hi_ref[...] * hj_ref[...]


_outer = pl.pallas_call(
    _outer_body,
    grid=(N // BI, N // BJ),
    in_specs=[
        pl.BlockSpec((BI, 1), lambda i, j: (i, 0)),
        pl.BlockSpec((1, BJ), lambda i, j: (0, j)),
    ],
    out_specs=pl.BlockSpec((BI, BJ), lambda i, j: (i, j)),
    out_shape=jax.ShapeDtypeStruct((N, N), jnp.float32),
)


@jax.jit
def kernel(node_feats, edge_idx, eps, W1, b1, W2, b2, W3, b3):
    src = jnp.concatenate(
        [edge_idx[0], jnp.zeros((E_PAD - E,), jnp.int32)]).reshape(TOT_CHUNKS, CHUNK)
    dst = jnp.concatenate(
        [edge_idx[1], jnp.full((E_PAD - E,), N, jnp.int32)]).reshape(TOT_CHUNKS, CHUNK)
    zeros = jnp.zeros((CHUNK, D), jnp.float32)

    partials = _sc_aggregate()(node_feats, src, dst, zeros)

    h = _mlp(node_feats, partials, partials,
             (1.0 * eps).reshape(1, 1),
             W1.T, b1.reshape(1, 16),
             W2.T, b2.reshape(1, 16),
             W3.T, b3.reshape(1, 1))

    return _outer(h, h.reshape(1, N))
